# baseline (speedup 1.0000x reference)
import functools

import jax
import jax.numpy as jnp
from jax import lax
from jax.experimental import pallas as pl
from jax.experimental.pallas import tpu as pltpu
from jax.experimental.pallas import tpu_sc as plsc

VOCAB = 100000
DIM = 128
BATCH = 4096
SEQ = 50
IDS_STRIDE = 128

NC = 2
NS = 16
NW = NC * NS
NB_W = BATCH // NW
G = 4
NG = NB_W // G
MG = 2
NSLOT = 2 * MG

_mesh = plsc.VectorSubcoreMesh(core_axis_name="c", subcore_axis_name="s")


@functools.partial(
    pl.kernel,
    mesh=_mesh,
    out_type=jax.ShapeDtypeStruct((BATCH, SEQ, DIM), jnp.float32),
    scratch_types=[
        pltpu.VMEM((NB_W, IDS_STRIDE), jnp.int32),
        pltpu.VMEM((NSLOT, G, SEQ, DIM), jnp.float32),
        pltpu.SemaphoreType.DMA,
        pltpu.SemaphoreType.DMA,
    ],
    compiler_params=pltpu.CompilerParams(use_tc_tiling_on_sc=True),
)
def _embed(ids_hbm, table_hbm, out_hbm, idx_v, slots, gsem, ssem):
    wid = lax.axis_index("s") * NC + lax.axis_index("c")
    bbase = wid * NB_W
    pltpu.sync_copy(ids_hbm.at[pl.ds(bbase, NB_W)], idx_v)

    def gather_group(g, b):
        for k in range(G):
            pltpu.async_copy(
                table_hbm.at[idx_v.at[g * G + k, pl.ds(0, SEQ)]],
                slots.at[b, k],
                gsem,
            )

    def wait_gather_group(b):
        for k in range(G):
            pltpu.make_async_copy(
                table_hbm.at[idx_v.at[0, pl.ds(0, SEQ)]], slots.at[b, k], gsem
            ).wait()

    def scatter_group(g, b):
        pltpu.async_copy(slots.at[b], out_hbm.at[pl.ds(bbase + g * G, G)], ssem)

    def wait_scatter():
        pltpu.make_async_copy(slots.at[0], out_hbm.at[pl.ds(bbase, G)], ssem).wait()

    for b in range(MG):
        gather_group(b, b)
    for g in range(MG):
        wait_gather_group(g)
        scatter_group(g, g)
        gather_group(g + MG, (g + MG) % NSLOT)
    def body(g, carry):
        b = lax.rem(g, NSLOT)
        wait_gather_group(b)
        scatter_group(g, b)
        wait_scatter()
        gather_group(g + MG, lax.rem(g + MG, NSLOT))
        return carry

    lax.fori_loop(MG, NG - MG, body, 0)
    for g in range(NG - MG, NG):
        wait_gather_group(g % NSLOT)
        scatter_group(g, g % NSLOT)
    for _ in range(NSLOT):
        wait_scatter()


def kernel(token_ids, table):
    ids = jnp.pad(token_ids.astype(jnp.int32), ((0, 0), (0, IDS_STRIDE - SEQ)))
    return _embed(ids, table)
